# Initial kernel scaffold; baseline (speedup 1.0000x reference)
import jax
import jax.numpy as jnp
from jax.experimental import pallas as pl


def kernel(x, edge_index, edge_attr, Wn1, We1, Wa1, Wen1, Weu1, Wn2, We2, Wa2, Wen2, Weu2):
    raise NotImplementedError("write your pallas kernel here")


# speedup vs baseline: 4.3413x; 4.3413x over previous
import functools

import jax
import jax.numpy as jnp
from jax import lax
from jax.experimental import pallas as pl
from jax.experimental.pallas import tpu as pltpu
from jax.experimental.pallas import tpu_sc as plsc

N = 10000
E = 160000
D_NODE = 128
D_EDGE = 16
HID = 64
HEADS = 2

NC = 2
NS = 16
L = 16

NP = 10240
RPS = NP // NS
C = 128
EPW = 5376
EP = EPW * NC * NS
NCH = EPW // C

f32 = jnp.float32
i32 = jnp.int32


def _splat(v):
    return jnp.full((L,), v, i32)


def _tc_node_body(x_ref, b_ref, u_ref, s_ref):
    r = jnp.dot(x_ref[...], b_ref[...], preferred_element_type=f32)
    u_ref[...] = r[:, :128]
    s_ref[...] = r[:, 128:132]


def _tc_node(x_pad, bmat):
    k = x_pad.shape[1]
    return pl.pallas_call(
        _tc_node_body,
        grid=(NP // 256,),
        in_specs=[
            pl.BlockSpec((256, k), lambda i: (i, 0)),
            pl.BlockSpec((k, 256), lambda i: (0, 0)),
        ],
        out_specs=[
            pl.BlockSpec((256, 128), lambda i: (i, 0)),
            pl.BlockSpec((256, 4), lambda i: (i, 0)),
        ],
        out_shape=[
            jax.ShapeDtypeStruct((NP, 128), f32),
            jax.ShapeDtypeStruct((NP, 4), f32),
        ],
    )(x_pad, bmat)


def _tc_edge_body(ea_ref, b_ref, v1_ref, v2_ref, ae_ref, r_ref):
    r = jnp.dot(ea_ref[...], b_ref[...], preferred_element_type=f32)
    v1_ref[...] = r[:, :128]
    v2_ref[...] = r[:, 128:256]
    ae_ref[...] = r[:, 256:264]
    r_ref[...] = r[:, 264:328]


def _tc_edge(ea_pad, bmat):
    return pl.pallas_call(
        _tc_edge_body,
        grid=(EP // 512,),
        in_specs=[
            pl.BlockSpec((512, D_EDGE), lambda i: (i, 0)),
            pl.BlockSpec((D_EDGE, 384), lambda i: (0, 0)),
        ],
        out_specs=[
            pl.BlockSpec((512, 128), lambda i: (i, 0)),
            pl.BlockSpec((512, 128), lambda i: (i, 0)),
            pl.BlockSpec((512, 8), lambda i: (i, 0)),
            pl.BlockSpec((512, 64), lambda i: (i, 0)),
        ],
        out_shape=[
            jax.ShapeDtypeStruct((EP, 128), f32),
            jax.ShapeDtypeStruct((EP, 128), f32),
            jax.ShapeDtypeStruct((EP, 8), f32),
            jax.ShapeDtypeStruct((EP, 64), f32),
        ],
    )(ea_pad, bmat)


def _elu(v):
    return jnp.where(v > 0, v, jnp.exp(v) - 1.0)


def _tc_mid_body(p0_ref, p1_ref, b_ref, u_ref, s_ref):
    ne = p0_ref[0] + p1_ref[0]
    x1 = _elu(ne)
    r = jnp.dot(x1, b_ref[...], preferred_element_type=f32)
    u_ref[...] = r[:, :128]
    s_ref[...] = r[:, 128:132]


def _tc_mid(ne0, ne1, bmat):
    return pl.pallas_call(
        _tc_mid_body,
        grid=(NP // 256,),
        in_specs=[
            pl.BlockSpec((1, 256, HID), lambda i: (0, i, 0)),
            pl.BlockSpec((1, 256, HID), lambda i: (0, i, 0)),
            pl.BlockSpec((HID, 256), lambda i: (0, 0)),
        ],
        out_specs=[
            pl.BlockSpec((256, 128), lambda i: (i, 0)),
            pl.BlockSpec((256, 4), lambda i: (i, 0)),
        ],
        out_shape=[
            jax.ShapeDtypeStruct((NP, 128), f32),
            jax.ShapeDtypeStruct((NP, 4), f32),
        ],
    )(ne0[None], ne1[None], bmat)


def _tc_fin_body(p0_ref, p1_ref, b_ref, x_ref, pt_ref, qt_ref):
    ne = p0_ref[0] + p1_ref[0]
    x_ref[...] = _elu(ne)
    r = jnp.dot(ne, b_ref[...], preferred_element_type=f32)
    pt_ref[...] = r[:, :64]
    qt_ref[...] = r[:, 64:128]


def _tc_fin(ne0, ne1, bmat):
    return pl.pallas_call(
        _tc_fin_body,
        grid=(NP // 256,),
        in_specs=[
            pl.BlockSpec((1, 256, HID), lambda i: (0, i, 0)),
            pl.BlockSpec((1, 256, HID), lambda i: (0, i, 0)),
            pl.BlockSpec((HID, 128), lambda i: (0, 0)),
        ],
        out_specs=[
            pl.BlockSpec((256, HID), lambda i: (i, 0)),
            pl.BlockSpec((256, HID), lambda i: (i, 0)),
            pl.BlockSpec((256, HID), lambda i: (i, 0)),
        ],
        out_shape=[
            jax.ShapeDtypeStruct((NP, HID), f32),
            jax.ShapeDtypeStruct((NP, HID), f32),
            jax.ShapeDtypeStruct((NP, HID), f32),
        ],
    )(ne0[None], ne1[None], bmat)


def _make_p1(mesh, acol):
    @functools.partial(
        pl.kernel,
        out_type=[
            jax.ShapeDtypeStruct((EP, 2), f32),
            jax.ShapeDtypeStruct((NP, 2), f32),
            jax.ShapeDtypeStruct((NP, 2), f32),
        ],
        mesh=mesh,
        compiler_params=pltpu.CompilerParams(needs_layout_passes=False, use_tc_tiling_on_sc=False),
        scratch_types=[
            pltpu.VMEM((NP * 4,), f32),
            pltpu.VMEM((C,), i32),
            pltpu.VMEM((C,), i32),
            pltpu.VMEM((C, 8), f32),
            pltpu.VMEM((C, 2), f32),
            pltpu.VMEM_SHARED((NP, 2), f32),
        ],
    )
    def p1(src_h, dst_h, ae_h, sct_h, z2_h, ex_h, d0_h, d1_h,
           sctv, srcv, dstv, aev, exv, dsp):
        cidx = lax.axis_index("c")
        sidx = lax.axis_index("s")
        wid = sidx * NC + cidx
        pltpu.sync_copy(sct_h, sctv)
        pltpu.sync_copy(z2_h, dsp.at[pl.ds(sidx * RPS, RPS)])
        plsc.subcore_barrier()

        def chunk(ci, _):
            base = wid * EPW + ci * C
            pltpu.sync_copy(src_h.at[pl.ds(base, C)], srcv)
            pltpu.sync_copy(dst_h.at[pl.ds(base, C)], dstv)
            pltpu.sync_copy(ae_h.at[pl.ds(base, C)], aev)
            for g in range(C // L):
                rows = lax.iota(i32, L) + g * L
                isrc = srcv[pl.ds(g * L, L)]
                idst = dstv[pl.ds(g * L, L)]
                for h in range(HEADS):
                    ai = plsc.load_gather(sctv, [idst * 4 + h])
                    aj = plsc.load_gather(sctv, [isrc * 4 + (2 + h)])
                    ae = plsc.load_gather(aev, [rows, _splat(acol + h)])
                    al = ai + aj + ae
                    al = jnp.where(al > 0, al, 0.01 * al)
                    plsc.store_scatter(exv, [rows, _splat(h)], jnp.exp(al))
            pltpu.sync_copy(exv, ex_h.at[pl.ds(base, C)])
            pltpu.sync_copy(exv, dsp.at[dstv], add=True)
            return 0

        lax.fori_loop(0, NCH, chunk, 0)
        plsc.subcore_barrier()
        rs = pl.ds(sidx * RPS, RPS)

        @pl.when(cidx == 0)
        def _():
            pltpu.sync_copy(dsp.at[rs], d0_h.at[rs])

        @pl.when(cidx == 1)
        def _():
            pltpu.sync_copy(dsp.at[rs], d1_h.at[rs])

    return p1


def _make_p2(mesh):
  @functools.partial(
    pl.kernel,
    out_type=[
        jax.ShapeDtypeStruct((NP, HID), f32),
        jax.ShapeDtypeStruct((NP, HID), f32),
    ],
    mesh=mesh,
    compiler_params=pltpu.CompilerParams(needs_layout_passes=False, use_tc_tiling_on_sc=False),
    scratch_types=[
        pltpu.VMEM((NP * 2,), f32),
        pltpu.VMEM((NP * 2,), f32),
        pltpu.VMEM((C,), i32),
        pltpu.VMEM((C,), i32),
        pltpu.VMEM((C, 2), f32),
        pltpu.VMEM((C, 2), f32),
        pltpu.VMEM((C, 128), f32),
        pltpu.VMEM((C, 128), f32),
        pltpu.VMEM((C, HID), f32),
        pltpu.VMEM_SHARED((NP, HID), f32),
        pltpu.SemaphoreType.DMA,
    ],
  )
  def _p2(src_h, dst_h, ex_h, d0_h, d1_h, u_h, v_h, z64_h, o0_h, o1_h,
          pa, pb, srcv, dstv, exv, wv, ub, vb, msgv, accsp, sem):
    cidx = lax.axis_index("c")
    sidx = lax.axis_index("s")
    wid = sidx * NC + cidx
    pltpu.sync_copy(d0_h, pa)
    pltpu.sync_copy(d1_h, pb)
    pltpu.sync_copy(z64_h, accsp.at[pl.ds(sidx * RPS, RPS)])
    plsc.subcore_barrier()

    def chunk(ci, _):
        base = wid * EPW + ci * C
        pltpu.sync_copy(src_h.at[pl.ds(base, C)], srcv)
        pltpu.sync_copy(dst_h.at[pl.ds(base, C)], dstv)
        pltpu.sync_copy(ex_h.at[pl.ds(base, C)], exv)
        pltpu.async_copy(u_h.at[srcv], ub, sem).wait()
        pltpu.sync_copy(v_h.at[pl.ds(base, C)], vb)
        for g in range(C // L):
            rows = lax.iota(i32, L) + g * L
            idst = dstv[pl.ds(g * L, L)]
            for h in range(HEADS):
                d = (plsc.load_gather(pa, [idst * 2 + h])
                     + plsc.load_gather(pb, [idst * 2 + h]))
                ex = plsc.load_gather(exv, [rows, _splat(h)])
                w = (0.5 * ex) / (d + 1e-16)
                plsc.store_scatter(wv, [rows, _splat(h)], w)
        for e in range(C):
            w0 = plsc.load_gather(wv, [_splat(e), _splat(0)])
            w1 = plsc.load_gather(wv, [_splat(e), _splat(1)])
            for q in range(HID // L):
                c0 = q * L
                m = (w0 * (ub[e, pl.ds(c0, L)] + vb[e, pl.ds(c0, L)])
                     + w1 * (ub[e, pl.ds(HID + c0, L)] + vb[e, pl.ds(HID + c0, L)]))
                msgv[e, pl.ds(c0, L)] = m
        pltpu.sync_copy(msgv, accsp.at[dstv], add=True)
        return 0

    lax.fori_loop(0, NCH, chunk, 0)
    plsc.subcore_barrier()
    rs = pl.ds(sidx * RPS, RPS)

    @pl.when(cidx == 0)
    def _():
        pltpu.sync_copy(accsp.at[rs], o0_h.at[rs])

    @pl.when(cidx == 1)
    def _():
        pltpu.sync_copy(accsp.at[rs], o1_h.at[rs])

  return _p2


def _make_p3(mesh):
  @functools.partial(
    pl.kernel,
    out_type=jax.ShapeDtypeStruct((EP, HID), f32),
    mesh=mesh,
    compiler_params=pltpu.CompilerParams(needs_layout_passes=False, use_tc_tiling_on_sc=False),
    scratch_types=[
        pltpu.VMEM((C,), i32),
        pltpu.VMEM((C,), i32),
        pltpu.VMEM((C, HID), f32),
        pltpu.VMEM((C, HID), f32),
        pltpu.VMEM((C, HID), f32),
        pltpu.VMEM((C, HID), f32),
        pltpu.SemaphoreType.DMA,
    ],
  )
  def _p3(src_h, dst_h, p_h, q_h, r_h, eo_h, srcv, dstv, pbuf, qbuf, rbuf, ob, sem):
    cidx = lax.axis_index("c")
    sidx = lax.axis_index("s")
    wid = sidx * NC + cidx

    def chunk(ci, _):
        base = wid * EPW + ci * C
        pltpu.sync_copy(src_h.at[pl.ds(base, C)], srcv)
        pltpu.sync_copy(dst_h.at[pl.ds(base, C)], dstv)
        ca = pltpu.async_copy(p_h.at[srcv], pbuf, sem)
        cb = pltpu.async_copy(q_h.at[dstv], qbuf, sem)
        pltpu.sync_copy(r_h.at[pl.ds(base, C)], rbuf)
        ca.wait()
        cb.wait()
        for e in range(C):
            for q in range(HID // L):
                sl = pl.ds(q * L, L)
                v = pbuf[e, sl] + qbuf[e, sl] + rbuf[e, sl]
                ob[e, sl] = jnp.maximum(v, 0.0)
        pltpu.sync_copy(ob, eo_h.at[pl.ds(base, C)])
        return 0

    lax.fori_loop(0, NCH, chunk, 0)

  return _p3


@functools.lru_cache(maxsize=1)
def _sc_kernels():
    mesh = plsc.VectorSubcoreMesh(core_axis_name="c", subcore_axis_name="s",
                                  num_cores=NC, num_subcores=NS)
    return (_make_p1(mesh, 0), _make_p1(mesh, 2), _make_p2(mesh),
            _make_p3(mesh))


def _node_mat(Wn, Wen, Wa, din):
    wai, waj = Wa[0, :HID], Wa[0, HID:2 * HID]
    wen_a = Wen[:, :HID]
    cols = []
    for h in range(HEADS):
        wn_h = Wn[h * HID:(h + 1) * HID, :]
        cols.append(wn_h.T @ wen_a.T)
    sc = []
    for h in range(HEADS):
        wn_h = Wn[h * HID:(h + 1) * HID, :]
        sc.append((wn_h.T @ wai)[:, None])
    for h in range(HEADS):
        wn_h = Wn[h * HID:(h + 1) * HID, :]
        sc.append((wn_h.T @ waj)[:, None])
    pad = jnp.zeros((din, 256 - 2 * HID - 4), f32)
    return jnp.concatenate(cols + sc + [pad], axis=1)


def _edge_mat(We1, Wen1, Wa1, We2, Wen2, Wa2, Weu2):
    cols = []
    for We, Wen in ((We1, Wen1), (We2, Wen2)):
        wen_b = Wen[:, HID:]
        for h in range(HEADS):
            we_h = We[h * HID:(h + 1) * HID, :]
            cols.append(we_h.T @ wen_b.T)
    ae = []
    for We, Wa in ((We1, Wa1), (We2, Wa2)):
        wae = Wa[0, 2 * HID:]
        for h in range(HEADS):
            we_h = We[h * HID:(h + 1) * HID, :]
            ae.append((we_h.T @ wae)[:, None])
    ae.append(jnp.zeros((D_EDGE, 4), f32))
    we2_mean = 0.5 * (We2[:HID, :] + We2[HID:, :])
    rmat = we2_mean.T @ Weu2[:, 2 * HID:].T
    pad = jnp.zeros((D_EDGE, 384 - 256 - 8 - HID), f32)
    return jnp.concatenate(cols + ae + [rmat, pad], axis=1)


def kernel(x, edge_index, edge_attr, Wn1, We1, Wa1, Wen1, Weu1,
           Wn2, We2, Wa2, Wen2, Weu2):
    src = edge_index[0].astype(i32)
    dst = edge_index[1].astype(i32)
    loop = jnp.arange(N, dtype=i32)
    padi = jnp.full((EP - E - N,), N, i32)
    srcp = jnp.concatenate([src, loop, padi])
    dstp = jnp.concatenate([dst, loop, padi])
    x_pad = jnp.pad(x, ((0, NP - N), (0, 0)))
    ea_pad = jnp.pad(edge_attr, ((0, EP - E), (0, 0)))
    z2 = jnp.zeros((RPS, 2), f32)
    z64 = jnp.zeros((RPS, HID), f32)

    bn1 = _node_mat(Wn1, Wen1, Wa1, D_NODE)
    bn2 = _node_mat(Wn2, Wen2, Wa2, HID)
    be = _edge_mat(We1, Wen1, Wa1, We2, Wen2, Wa2, Weu2)
    wab = jnp.concatenate([Weu2[:, :HID].T, Weu2[:, HID:2 * HID].T], axis=1)

    p1l1, p1l2, p2, p3 = _sc_kernels()

    u1, sc1 = _tc_node(x_pad, bn1)
    v1, v2, ae8, rm = _tc_edge(ea_pad, be)

    ex1, d10, d11 = p1l1(srcp, dstp, ae8, sc1.reshape(-1), z2)
    ne10, ne11 = p2(srcp, dstp, ex1, d10.reshape(-1), d11.reshape(-1), u1, v1, z64)
    u2, sc2 = _tc_mid(ne10, ne11, bn2)
    ex2, d20, d21 = p1l2(srcp, dstp, ae8, sc2.reshape(-1), z2)
    ne20, ne21 = p2(srcp, dstp, ex2, d20.reshape(-1), d21.reshape(-1), u2, v2, z64)
    xo, pt, qt = _tc_fin(ne20, ne21, wab)
    eo = p3(srcp, dstp, pt, qt, rm)

    return xo[:N], eo[:E]


# speedup vs baseline: 5.2084x; 1.1997x over previous
import functools

import jax
import jax.numpy as jnp
from jax import lax
from jax.experimental import pallas as pl
from jax.experimental.pallas import tpu as pltpu
from jax.experimental.pallas import tpu_sc as plsc

N = 10000
E = 160000
D_NODE = 128
D_EDGE = 16
HID = 64
HEADS = 2

NC = 2
NS = 16
L = 16

NP = 10240
RPS = NP // NS
C = 96
EPW = 5376
EP = EPW * NC * NS
NCH = EPW // C

f32 = jnp.float32
i32 = jnp.int32


def _splat(v):
    return jnp.full((L,), v, i32)


def _tc_node_body(x_ref, b_ref, u_ref, s_ref):
    r = jnp.dot(x_ref[...], b_ref[...], preferred_element_type=f32)
    u_ref[...] = r[:, :128]
    s_ref[...] = r[:, 128:132]


def _tc_node(x_pad, bmat):
    k = x_pad.shape[1]
    return pl.pallas_call(
        _tc_node_body,
        grid=(NP // 256,),
        in_specs=[
            pl.BlockSpec((256, k), lambda i: (i, 0)),
            pl.BlockSpec((k, 256), lambda i: (0, 0)),
        ],
        out_specs=[
            pl.BlockSpec((256, 128), lambda i: (i, 0)),
            pl.BlockSpec((256, 4), lambda i: (i, 0)),
        ],
        out_shape=[
            jax.ShapeDtypeStruct((NP, 128), f32),
            jax.ShapeDtypeStruct((NP, 4), f32),
        ],
    )(x_pad, bmat)


def _tc_edge_body(ea_ref, b_ref, v1_ref, v2_ref, ae_ref, r_ref):
    r = jnp.dot(ea_ref[...], b_ref[...], preferred_element_type=f32)
    v1_ref[...] = r[:, :128]
    v2_ref[...] = r[:, 128:256]
    ae_ref[...] = r[:, 256:264]
    r_ref[...] = r[:, 264:328]


def _tc_edge(ea_pad, bmat):
    return pl.pallas_call(
        _tc_edge_body,
        grid=(EP // 512,),
        in_specs=[
            pl.BlockSpec((512, D_EDGE), lambda i: (i, 0)),
            pl.BlockSpec((D_EDGE, 384), lambda i: (0, 0)),
        ],
        out_specs=[
            pl.BlockSpec((512, 128), lambda i: (i, 0)),
            pl.BlockSpec((512, 128), lambda i: (i, 0)),
            pl.BlockSpec((512, 8), lambda i: (i, 0)),
            pl.BlockSpec((512, 64), lambda i: (i, 0)),
        ],
        out_shape=[
            jax.ShapeDtypeStruct((EP, 128), f32),
            jax.ShapeDtypeStruct((EP, 128), f32),
            jax.ShapeDtypeStruct((EP, 8), f32),
            jax.ShapeDtypeStruct((EP, 64), f32),
        ],
    )(ea_pad, bmat)


def _elu(v):
    return jnp.where(v > 0, v, jnp.exp(v) - 1.0)


def _tc_mid_body(p0_ref, p1_ref, b_ref, u_ref, s_ref):
    ne = p0_ref[0] + p1_ref[0]
    x1 = _elu(ne)
    r = jnp.dot(x1, b_ref[...], preferred_element_type=f32)
    u_ref[...] = r[:, :128]
    s_ref[...] = r[:, 128:132]


def _tc_mid(ne0, ne1, bmat):
    return pl.pallas_call(
        _tc_mid_body,
        grid=(NP // 256,),
        in_specs=[
            pl.BlockSpec((1, 256, HID), lambda i: (0, i, 0)),
            pl.BlockSpec((1, 256, HID), lambda i: (0, i, 0)),
            pl.BlockSpec((HID, 256), lambda i: (0, 0)),
        ],
        out_specs=[
            pl.BlockSpec((256, 128), lambda i: (i, 0)),
            pl.BlockSpec((256, 4), lambda i: (i, 0)),
        ],
        out_shape=[
            jax.ShapeDtypeStruct((NP, 128), f32),
            jax.ShapeDtypeStruct((NP, 4), f32),
        ],
    )(ne0[None], ne1[None], bmat)


def _tc_fin_body(p0_ref, p1_ref, b_ref, x_ref, pt_ref, qt_ref):
    ne = p0_ref[0] + p1_ref[0]
    x_ref[...] = _elu(ne)
    r = jnp.dot(ne, b_ref[...], preferred_element_type=f32)
    pt_ref[...] = r[:, :64]
    qt_ref[...] = r[:, 64:128]


def _tc_fin(ne0, ne1, bmat):
    return pl.pallas_call(
        _tc_fin_body,
        grid=(NP // 256,),
        in_specs=[
            pl.BlockSpec((1, 256, HID), lambda i: (0, i, 0)),
            pl.BlockSpec((1, 256, HID), lambda i: (0, i, 0)),
            pl.BlockSpec((HID, 128), lambda i: (0, 0)),
        ],
        out_specs=[
            pl.BlockSpec((256, HID), lambda i: (i, 0)),
            pl.BlockSpec((256, HID), lambda i: (i, 0)),
            pl.BlockSpec((256, HID), lambda i: (i, 0)),
        ],
        out_shape=[
            jax.ShapeDtypeStruct((NP, HID), f32),
            jax.ShapeDtypeStruct((NP, HID), f32),
            jax.ShapeDtypeStruct((NP, HID), f32),
        ],
    )(ne0[None], ne1[None], bmat)


def _make_p1(mesh, acol):
    @functools.partial(
        pl.kernel,
        out_type=[
            jax.ShapeDtypeStruct((EP, 2), f32),
            jax.ShapeDtypeStruct((NP, 2), f32),
            jax.ShapeDtypeStruct((NP, 2), f32),
        ],
        mesh=mesh,
        compiler_params=pltpu.CompilerParams(needs_layout_passes=False, use_tc_tiling_on_sc=False),
        scratch_types=[
            pltpu.VMEM((NP * 4,), f32),
            pltpu.VMEM((C,), i32),
            pltpu.VMEM((C,), i32),
            pltpu.VMEM((C, 8), f32),
            pltpu.VMEM((C, 2), f32),
            pltpu.VMEM_SHARED((NP, 2), f32),
        ],
    )
    def p1(src_h, dst_h, ae_h, sct_h, z2_h, ex_h, d0_h, d1_h,
           sctv, srcv, dstv, aev, exv, dsp):
        cidx = lax.axis_index("c")
        sidx = lax.axis_index("s")
        wid = sidx * NC + cidx
        pltpu.sync_copy(sct_h, sctv)
        pltpu.sync_copy(z2_h, dsp.at[pl.ds(sidx * RPS, RPS)])
        plsc.subcore_barrier()

        def chunk(ci, _):
            base = wid * EPW + ci * C
            pltpu.sync_copy(src_h.at[pl.ds(base, C)], srcv)
            pltpu.sync_copy(dst_h.at[pl.ds(base, C)], dstv)
            pltpu.sync_copy(ae_h.at[pl.ds(base, C)], aev)
            for g in range(C // L):
                rows = lax.iota(i32, L) + g * L
                isrc = srcv[pl.ds(g * L, L)]
                idst = dstv[pl.ds(g * L, L)]
                for h in range(HEADS):
                    ai = plsc.load_gather(sctv, [idst * 4 + h])
                    aj = plsc.load_gather(sctv, [isrc * 4 + (2 + h)])
                    ae = plsc.load_gather(aev, [rows, _splat(acol + h)])
                    al = ai + aj + ae
                    al = jnp.where(al > 0, al, 0.01 * al)
                    plsc.store_scatter(exv, [rows, _splat(h)], jnp.exp(al))
            pltpu.sync_copy(exv, ex_h.at[pl.ds(base, C)])
            pltpu.sync_copy(exv, dsp.at[dstv], add=True)
            return 0

        lax.fori_loop(0, NCH, chunk, 0)
        plsc.subcore_barrier()
        rs = pl.ds(sidx * RPS, RPS)

        @pl.when(cidx == 0)
        def _():
            pltpu.sync_copy(dsp.at[rs], d0_h.at[rs])

        @pl.when(cidx == 1)
        def _():
            pltpu.sync_copy(dsp.at[rs], d1_h.at[rs])

    return p1


def _make_p2(mesh):
  @functools.partial(
    pl.kernel,
    out_type=[
        jax.ShapeDtypeStruct((NP, HID), f32),
        jax.ShapeDtypeStruct((NP, HID), f32),
    ],
    mesh=mesh,
    compiler_params=pltpu.CompilerParams(needs_layout_passes=False, use_tc_tiling_on_sc=False),
    scratch_types=[
        pltpu.VMEM((NP * 2,), f32),
        pltpu.VMEM((EPW,), i32),
        pltpu.VMEM((EPW,), i32),
        pltpu.VMEM((512,), f32),
        pltpu.VMEM((C, 2), f32),
        pltpu.VMEM((C, 2), f32),
        pltpu.VMEM((C, 2), f32),
        pltpu.VMEM((C, 128), f32),
        pltpu.VMEM((C, 128), f32),
        pltpu.VMEM((C, 128), f32),
        pltpu.VMEM((C, 128), f32),
        pltpu.VMEM((C, HID), f32),
        pltpu.VMEM((C,), i32),
        pltpu.VMEM_SHARED((NP, HID), f32),
        pltpu.SemaphoreType.DMA,
        pltpu.SemaphoreType.DMA,
    ],
  )
  def _p2(src_h, dst_h, ex_h, d0_h, d1_h, u_h, v_h, z64_h, o0_h, o1_h,
          pd, srca, dsta, tmpv, exv0, exv1, wv, ub0, ub1, vb0, vb1,
          msgv, dstv, accsp, semg0, semg1):
    cidx = lax.axis_index("c")
    sidx = lax.axis_index("s")
    wid = sidx * NC + cidx
    ebase = wid * EPW
    pltpu.sync_copy(d0_h, pd)
    pltpu.sync_copy(src_h.at[pl.ds(ebase, EPW)], srca)
    pltpu.sync_copy(dst_h.at[pl.ds(ebase, EPW)], dsta)
    pltpu.sync_copy(z64_h, accsp.at[pl.ds(sidx * RPS, RPS)])

    def dadd(i, _):
        pltpu.sync_copy(d1_h.at[pl.ds(i * 512, 512)], tmpv)
        for j in range(512 // L):
            sl = pl.ds(i * 512 + j * L, L)
            pd[sl] = pd[sl] + tmpv[pl.ds(j * L, L)]
        return 0

    lax.fori_loop(0, (NP * 2) // 512, dadd, 0)
    plsc.subcore_barrier()

    exv = (exv0, exv1)
    ub = (ub0, ub1)
    vb = (vb0, vb1)
    semg = (semg0, semg1)

    def issue(b, ci):
        pltpu.async_copy(u_h.at[srca.at[pl.ds(ci * C, C)]], ub[b], semg[b])
        pltpu.async_copy(v_h.at[pl.ds(ebase + ci * C, C)], vb[b], semg[b])
        pltpu.async_copy(ex_h.at[pl.ds(ebase + ci * C, C)], exv[b], semg[b])

    def wait(b, ci):
        pltpu.make_async_copy(u_h.at[srca.at[pl.ds(ci * C, C)]], ub[b], semg[b]).wait()
        pltpu.make_async_copy(v_h.at[pl.ds(ebase + ci * C, C)], vb[b], semg[b]).wait()
        pltpu.make_async_copy(ex_h.at[pl.ds(ebase + ci * C, C)], exv[b], semg[b]).wait()

    issue(0, 0)

    def pair(p, _):
        for b in range(2):
            ci = 2 * p + b

            @pl.when(ci + 1 < NCH)
            def _():
                issue(1 - b, ci + 1)

            wait(b, ci)
            for g in range(C // L):
                ii = g * L
                rows = lax.iota(i32, L) + ii
                idst = dsta[pl.ds(ci * C + ii, L)]
                dstv[pl.ds(ii, L)] = idst
                for h in range(HEADS):
                    d = plsc.load_gather(pd, [idst * 2 + h])
                    ex = plsc.load_gather(exv[b], [rows, _splat(h)])
                    w = (0.5 * ex) / (d + 1e-16)
                    plsc.store_scatter(wv, [rows, _splat(h)], w)
            for e in range(C):
                w0 = plsc.load_gather(wv, [_splat(e), _splat(0)])
                w1 = plsc.load_gather(wv, [_splat(e), _splat(1)])
                for q in range(HID // L):
                    c0 = q * L
                    m = (w0 * (ub[b][e, pl.ds(c0, L)] + vb[b][e, pl.ds(c0, L)])
                         + w1 * (ub[b][e, pl.ds(HID + c0, L)] + vb[b][e, pl.ds(HID + c0, L)]))
                    msgv[e, pl.ds(c0, L)] = m
            pltpu.sync_copy(msgv, accsp.at[dstv], add=True)
        return 0

    lax.fori_loop(0, NCH // 2, pair, 0)
    plsc.subcore_barrier()
    rs = pl.ds(sidx * RPS, RPS)

    @pl.when(cidx == 0)
    def _():
        pltpu.sync_copy(accsp.at[rs], o0_h.at[rs])

    @pl.when(cidx == 1)
    def _():
        pltpu.sync_copy(accsp.at[rs], o1_h.at[rs])

  return _p2


def _make_p3(mesh):
  @functools.partial(
    pl.kernel,
    out_type=jax.ShapeDtypeStruct((EP, HID), f32),
    mesh=mesh,
    compiler_params=pltpu.CompilerParams(needs_layout_passes=False, use_tc_tiling_on_sc=False),
    scratch_types=[
        pltpu.VMEM((EPW,), i32),
        pltpu.VMEM((EPW,), i32),
        pltpu.VMEM((C, HID), f32),
        pltpu.VMEM((C, HID), f32),
        pltpu.VMEM((C, HID), f32),
        pltpu.VMEM((C, HID), f32),
        pltpu.VMEM((C, HID), f32),
        pltpu.VMEM((C, HID), f32),
        pltpu.VMEM((C, HID), f32),
        pltpu.SemaphoreType.DMA,
        pltpu.SemaphoreType.DMA,
    ],
  )
  def _p3(src_h, dst_h, p_h, q_h, r_h, eo_h, srca, dsta,
          pbuf0, pbuf1, qbuf0, qbuf1, rbuf0, rbuf1, ob, semg0, semg1):
    cidx = lax.axis_index("c")
    sidx = lax.axis_index("s")
    wid = sidx * NC + cidx
    ebase = wid * EPW
    pltpu.sync_copy(src_h.at[pl.ds(ebase, EPW)], srca)
    pltpu.sync_copy(dst_h.at[pl.ds(ebase, EPW)], dsta)

    pbuf = (pbuf0, pbuf1)
    qbuf = (qbuf0, qbuf1)
    rbuf = (rbuf0, rbuf1)
    semg = (semg0, semg1)

    def issue(b, ci):
        pltpu.async_copy(p_h.at[srca.at[pl.ds(ci * C, C)]], pbuf[b], semg[b])
        pltpu.async_copy(q_h.at[dsta.at[pl.ds(ci * C, C)]], qbuf[b], semg[b])
        pltpu.async_copy(r_h.at[pl.ds(ebase + ci * C, C)], rbuf[b], semg[b])

    def wait(b, ci):
        pltpu.make_async_copy(p_h.at[srca.at[pl.ds(ci * C, C)]], pbuf[b], semg[b]).wait()
        pltpu.make_async_copy(q_h.at[dsta.at[pl.ds(ci * C, C)]], qbuf[b], semg[b]).wait()
        pltpu.make_async_copy(r_h.at[pl.ds(ebase + ci * C, C)], rbuf[b], semg[b]).wait()

    issue(0, 0)

    def pair(p, _):
        for b in range(2):
            ci = 2 * p + b

            @pl.when(ci + 1 < NCH)
            def _():
                issue(1 - b, ci + 1)

            wait(b, ci)
            for e in range(C):
                for q in range(HID // L):
                    sl = pl.ds(q * L, L)
                    v = pbuf[b][e, sl] + qbuf[b][e, sl] + rbuf[b][e, sl]
                    ob[e, sl] = jnp.maximum(v, 0.0)
            pltpu.sync_copy(ob, eo_h.at[pl.ds(ebase + ci * C, C)])
        return 0

    lax.fori_loop(0, NCH // 2, pair, 0)

  return _p3


@functools.lru_cache(maxsize=1)
def _sc_kernels():
    mesh = plsc.VectorSubcoreMesh(core_axis_name="c", subcore_axis_name="s",
                                  num_cores=NC, num_subcores=NS)
    return (_make_p1(mesh, 0), _make_p1(mesh, 2), _make_p2(mesh),
            _make_p3(mesh))


def _node_mat(Wn, Wen, Wa, din):
    wai, waj = Wa[0, :HID], Wa[0, HID:2 * HID]
    wen_a = Wen[:, :HID]
    cols = []
    for h in range(HEADS):
        wn_h = Wn[h * HID:(h + 1) * HID, :]
        cols.append(wn_h.T @ wen_a.T)
    sc = []
    for h in range(HEADS):
        wn_h = Wn[h * HID:(h + 1) * HID, :]
        sc.append((wn_h.T @ wai)[:, None])
    for h in range(HEADS):
        wn_h = Wn[h * HID:(h + 1) * HID, :]
        sc.append((wn_h.T @ waj)[:, None])
    pad = jnp.zeros((din, 256 - 2 * HID - 4), f32)
    return jnp.concatenate(cols + sc + [pad], axis=1)


def _edge_mat(We1, Wen1, Wa1, We2, Wen2, Wa2, Weu2):
    cols = []
    for We, Wen in ((We1, Wen1), (We2, Wen2)):
        wen_b = Wen[:, HID:]
        for h in range(HEADS):
            we_h = We[h * HID:(h + 1) * HID, :]
            cols.append(we_h.T @ wen_b.T)
    ae = []
    for We, Wa in ((We1, Wa1), (We2, Wa2)):
        wae = Wa[0, 2 * HID:]
        for h in range(HEADS):
            we_h = We[h * HID:(h + 1) * HID, :]
            ae.append((we_h.T @ wae)[:, None])
    ae.append(jnp.zeros((D_EDGE, 4), f32))
    we2_mean = 0.5 * (We2[:HID, :] + We2[HID:, :])
    rmat = we2_mean.T @ Weu2[:, 2 * HID:].T
    pad = jnp.zeros((D_EDGE, 384 - 256 - 8 - HID), f32)
    return jnp.concatenate(cols + ae + [rmat, pad], axis=1)


def kernel(x, edge_index, edge_attr, Wn1, We1, Wa1, Wen1, Weu1,
           Wn2, We2, Wa2, Wen2, Weu2):
    src = edge_index[0].astype(i32)
    dst = edge_index[1].astype(i32)
    loop = jnp.arange(N, dtype=i32)
    padi = jnp.full((EP - E - N,), N, i32)
    srcp = jnp.concatenate([src, loop, padi])
    dstp = jnp.concatenate([dst, loop, padi])
    x_pad = jnp.pad(x, ((0, NP - N), (0, 0)))
    ea_pad = jnp.pad(edge_attr, ((0, EP - E), (0, 0)))
    z2 = jnp.zeros((RPS, 2), f32)
    z64 = jnp.zeros((RPS, HID), f32)

    bn1 = _node_mat(Wn1, Wen1, Wa1, D_NODE)
    bn2 = _node_mat(Wn2, Wen2, Wa2, HID)
    be = _edge_mat(We1, Wen1, Wa1, We2, Wen2, Wa2, Weu2)
    wab = jnp.concatenate([Weu2[:, :HID].T, Weu2[:, HID:2 * HID].T], axis=1)

    p1l1, p1l2, p2, p3 = _sc_kernels()

    u1, sc1 = _tc_node(x_pad, bn1)
    v1, v2, ae8, rm = _tc_edge(ea_pad, be)

    ex1, d10, d11 = p1l1(srcp, dstp, ae8, sc1.reshape(-1), z2)
    ne10, ne11 = p2(srcp, dstp, ex1, d10.reshape(-1), d11.reshape(-1), u1, v1, z64)
    u2, sc2 = _tc_mid(ne10, ne11, bn2)
    ex2, d20, d21 = p1l2(srcp, dstp, ae8, sc2.reshape(-1), z2)
    ne20, ne21 = p2(srcp, dstp, ex2, d20.reshape(-1), d21.reshape(-1), u2, v2, z64)
    xo, pt, qt = _tc_fin(ne20, ne21, wab)
    eo = p3(srcp, dstp, pt, qt, rm)

    return xo[:N], eo[:E]


# speedup vs baseline: 5.7212x; 1.0985x over previous
import functools

import jax
import jax.numpy as jnp
from jax import lax
from jax.experimental import pallas as pl
from jax.experimental.pallas import tpu as pltpu
from jax.experimental.pallas import tpu_sc as plsc

N = 10000
E = 160000
D_NODE = 128
D_EDGE = 16
HID = 64
HEADS = 2

NC = 2
NS = 16
L = 16

NP = 10240
RPS = NP // NS
C = 96
EPW = 5376
EP = EPW * NC * NS
NCH = EPW // C

f32 = jnp.float32
i32 = jnp.int32


def _splat(v):
    return jnp.full((L,), v, i32)


def _tc_node_body(x_ref, b_ref, u_ref, s_ref):
    r = jnp.dot(x_ref[...], b_ref[...], preferred_element_type=f32)
    u_ref[...] = r[:, :128]
    s_ref[...] = r[:, 128:132]


def _tc_node(x_pad, bmat):
    k = x_pad.shape[1]
    return pl.pallas_call(
        _tc_node_body,
        grid=(NP // 256,),
        in_specs=[
            pl.BlockSpec((256, k), lambda i: (i, 0)),
            pl.BlockSpec((k, 256), lambda i: (0, 0)),
        ],
        out_specs=[
            pl.BlockSpec((256, 128), lambda i: (i, 0)),
            pl.BlockSpec((256, 4), lambda i: (i, 0)),
        ],
        out_shape=[
            jax.ShapeDtypeStruct((NP, 128), f32),
            jax.ShapeDtypeStruct((NP, 4), f32),
        ],
    )(x_pad, bmat)


def _tc_edge_body(ea_ref, b_ref, v1_ref, v2_ref, ae_ref, r_ref):
    r = jnp.dot(ea_ref[...], b_ref[...], preferred_element_type=f32)
    v1_ref[...] = r[:, :128]
    v2_ref[...] = r[:, 128:256]
    ae_ref[...] = r[:, 256:264]
    r_ref[...] = r[:, 264:328]


def _tc_edge(ea_pad, bmat):
    return pl.pallas_call(
        _tc_edge_body,
        grid=(EP // 512,),
        in_specs=[
            pl.BlockSpec((512, D_EDGE), lambda i: (i, 0)),
            pl.BlockSpec((D_EDGE, 384), lambda i: (0, 0)),
        ],
        out_specs=[
            pl.BlockSpec((512, 128), lambda i: (i, 0)),
            pl.BlockSpec((512, 128), lambda i: (i, 0)),
            pl.BlockSpec((512, 8), lambda i: (i, 0)),
            pl.BlockSpec((512, 64), lambda i: (i, 0)),
        ],
        out_shape=[
            jax.ShapeDtypeStruct((EP, 128), f32),
            jax.ShapeDtypeStruct((EP, 128), f32),
            jax.ShapeDtypeStruct((EP, 8), f32),
            jax.ShapeDtypeStruct((EP, 64), f32),
        ],
    )(ea_pad, bmat)


def _elu(v):
    return jnp.where(v > 0, v, jnp.exp(v) - 1.0)


def _tc_mid_body(p0_ref, p1_ref, b_ref, u_ref, s_ref):
    ne = p0_ref[0] + p1_ref[0]
    x1 = _elu(ne)
    r = jnp.dot(x1, b_ref[...], preferred_element_type=f32)
    u_ref[...] = r[:, :128]
    s_ref[...] = r[:, 128:132]


def _tc_mid(ne0, ne1, bmat):
    return pl.pallas_call(
        _tc_mid_body,
        grid=(NP // 256,),
        in_specs=[
            pl.BlockSpec((1, 256, HID), lambda i: (0, i, 0)),
            pl.BlockSpec((1, 256, HID), lambda i: (0, i, 0)),
            pl.BlockSpec((HID, 256), lambda i: (0, 0)),
        ],
        out_specs=[
            pl.BlockSpec((256, 128), lambda i: (i, 0)),
            pl.BlockSpec((256, 4), lambda i: (i, 0)),
        ],
        out_shape=[
            jax.ShapeDtypeStruct((NP, 128), f32),
            jax.ShapeDtypeStruct((NP, 4), f32),
        ],
    )(ne0[None], ne1[None], bmat)


def _tc_fin_body(p0_ref, p1_ref, b_ref, x_ref, pt_ref, qt_ref):
    ne = p0_ref[0] + p1_ref[0]
    x_ref[...] = _elu(ne)
    r = jnp.dot(ne, b_ref[...], preferred_element_type=f32)
    pt_ref[...] = r[:, :64]
    qt_ref[...] = r[:, 64:128]


def _tc_fin(ne0, ne1, bmat):
    return pl.pallas_call(
        _tc_fin_body,
        grid=(NP // 256,),
        in_specs=[
            pl.BlockSpec((1, 256, HID), lambda i: (0, i, 0)),
            pl.BlockSpec((1, 256, HID), lambda i: (0, i, 0)),
            pl.BlockSpec((HID, 128), lambda i: (0, 0)),
        ],
        out_specs=[
            pl.BlockSpec((256, HID), lambda i: (i, 0)),
            pl.BlockSpec((256, HID), lambda i: (i, 0)),
            pl.BlockSpec((256, HID), lambda i: (i, 0)),
        ],
        out_shape=[
            jax.ShapeDtypeStruct((NP, HID), f32),
            jax.ShapeDtypeStruct((NP, HID), f32),
            jax.ShapeDtypeStruct((NP, HID), f32),
        ],
    )(ne0[None], ne1[None], bmat)


def _make_p1(mesh, acol):
    @functools.partial(
        pl.kernel,
        out_type=[
            jax.ShapeDtypeStruct((EP, 2), f32),
            jax.ShapeDtypeStruct((NP, 2), f32),
            jax.ShapeDtypeStruct((NP, 2), f32),
        ],
        mesh=mesh,
        compiler_params=pltpu.CompilerParams(needs_layout_passes=False, use_tc_tiling_on_sc=False),
        scratch_types=[
            pltpu.VMEM((NP * 4,), f32),
            pltpu.VMEM((EPW,), i32),
            pltpu.VMEM((EPW,), i32),
            pltpu.VMEM((EPW, 8), f32),
            pltpu.VMEM((C, 2), f32),
            pltpu.VMEM((C, 2), f32),
            pltpu.VMEM((C,), i32),
            pltpu.VMEM((C,), i32),
            pltpu.VMEM_SHARED((NP, 2), f32),
            pltpu.SemaphoreType.DMA,
            pltpu.SemaphoreType.DMA,
        ],
    )
    def p1(src_h, dst_h, ae_h, sct_h, z2_h, ex_h, d0_h, d1_h,
           sctv, srca, dsta, aev, exv0, exv1, dstv0, dstv1, dsp,
           sem0, sem1):
        cidx = lax.axis_index("c")
        sidx = lax.axis_index("s")
        wid = sidx * NC + cidx
        ebase = wid * EPW
        pltpu.sync_copy(sct_h, sctv)
        pltpu.sync_copy(src_h.at[pl.ds(ebase, EPW)], srca)
        pltpu.sync_copy(dst_h.at[pl.ds(ebase, EPW)], dsta)
        pltpu.sync_copy(ae_h.at[pl.ds(ebase, EPW)], aev)
        pltpu.sync_copy(z2_h, dsp.at[pl.ds(sidx * RPS, RPS)])
        plsc.subcore_barrier()

        def chunk(ci, _):
            for g in range(C // L):
                ii = g * L
                rows = lax.iota(i32, L) + ii
                gsl = pl.ds(ci * C + ii, L)
                isrc = srca[gsl]
                idst = dsta[gsl]
                dstv0[pl.ds(ii, L)] = idst
                grows = rows + ci * C
                for h in range(HEADS):
                    ai = plsc.load_gather(sctv, [idst * 4 + h])
                    aj = plsc.load_gather(sctv, [isrc * 4 + (2 + h)])
                    ae = plsc.load_gather(aev, [grows, _splat(acol + h)])
                    al = ai + aj + ae
                    al = jnp.where(al > 0, al, 0.01 * al)
                    plsc.store_scatter(exv0, [rows, _splat(h)], jnp.exp(al))
            pltpu.sync_copy(exv0, ex_h.at[pl.ds(ebase + ci * C, C)])
            pltpu.sync_copy(exv0, dsp.at[dstv0], add=True)
            return 0

        lax.fori_loop(0, NCH, chunk, 0)
        plsc.subcore_barrier()
        rs = pl.ds(sidx * RPS, RPS)

        @pl.when(cidx == 0)
        def _():
            pltpu.sync_copy(dsp.at[rs], d0_h.at[rs])

        @pl.when(cidx == 1)
        def _():
            pltpu.sync_copy(dsp.at[rs], d1_h.at[rs])

    return p1


def _make_p2(mesh):
  @functools.partial(
    pl.kernel,
    out_type=[
        jax.ShapeDtypeStruct((NP, HID), f32),
        jax.ShapeDtypeStruct((NP, HID), f32),
    ],
    mesh=mesh,
    compiler_params=pltpu.CompilerParams(needs_layout_passes=False, use_tc_tiling_on_sc=False),
    scratch_types=[
        pltpu.VMEM((NP * 2,), f32),
        pltpu.VMEM((EPW,), i32),
        pltpu.VMEM((EPW,), i32),
        pltpu.VMEM((512,), f32),
        pltpu.VMEM((C, 2), f32),
        pltpu.VMEM((C, 2), f32),
        pltpu.VMEM((C, 2), f32),
        pltpu.VMEM((C, 128), f32),
        pltpu.VMEM((C, 128), f32),
        pltpu.VMEM((C, 128), f32),
        pltpu.VMEM((C, 128), f32),
        pltpu.VMEM((C, HID), f32),
        pltpu.VMEM((C,), i32),
        pltpu.VMEM_SHARED((NP, HID), f32),
        pltpu.SemaphoreType.DMA,
        pltpu.SemaphoreType.DMA,
    ],
  )
  def _p2(src_h, dst_h, ex_h, d0_h, d1_h, u_h, v_h, z64_h, o0_h, o1_h,
          pd, srca, dsta, tmpv, exv0, exv1, wv, ub0, ub1, vb0, vb1,
          msgv, dstv, accsp, semg0, semg1):
    cidx = lax.axis_index("c")
    sidx = lax.axis_index("s")
    wid = sidx * NC + cidx
    ebase = wid * EPW
    pltpu.sync_copy(d0_h, pd)
    pltpu.sync_copy(src_h.at[pl.ds(ebase, EPW)], srca)
    pltpu.sync_copy(dst_h.at[pl.ds(ebase, EPW)], dsta)
    pltpu.sync_copy(z64_h, accsp.at[pl.ds(sidx * RPS, RPS)])

    def dadd(i, _):
        pltpu.sync_copy(d1_h.at[pl.ds(i * 512, 512)], tmpv)
        for j in range(512 // L):
            sl = pl.ds(i * 512 + j * L, L)
            pd[sl] = pd[sl] + tmpv[pl.ds(j * L, L)]
        return 0

    lax.fori_loop(0, (NP * 2) // 512, dadd, 0)
    plsc.subcore_barrier()

    exv = (exv0, exv1)
    ub = (ub0, ub1)
    vb = (vb0, vb1)
    semg = (semg0, semg1)

    def issue(b, ci):
        pltpu.async_copy(u_h.at[srca.at[pl.ds(ci * C, C)]], ub[b], semg[b])
        pltpu.async_copy(v_h.at[pl.ds(ebase + ci * C, C)], vb[b], semg[b])
        pltpu.async_copy(ex_h.at[pl.ds(ebase + ci * C, C)], exv[b], semg[b])

    def wait(b, ci):
        pltpu.make_async_copy(u_h.at[srca.at[pl.ds(ci * C, C)]], ub[b], semg[b]).wait()
        pltpu.make_async_copy(v_h.at[pl.ds(ebase + ci * C, C)], vb[b], semg[b]).wait()
        pltpu.make_async_copy(ex_h.at[pl.ds(ebase + ci * C, C)], exv[b], semg[b]).wait()

    issue(0, 0)

    def pair(p, _):
        for b in range(2):
            ci = 2 * p + b

            @pl.when(ci + 1 < NCH)
            def _():
                issue(1 - b, ci + 1)

            wait(b, ci)
            for g in range(C // L):
                ii = g * L
                rows = lax.iota(i32, L) + ii
                idst = dsta[pl.ds(ci * C + ii, L)]
                dstv[pl.ds(ii, L)] = idst
                for h in range(HEADS):
                    d = plsc.load_gather(pd, [idst * 2 + h])
                    ex = plsc.load_gather(exv[b], [rows, _splat(h)])
                    w = (0.5 * ex) / (d + 1e-16)
                    plsc.store_scatter(wv, [rows, _splat(h)], w)
            for e in range(C):
                w0 = plsc.load_gather(wv, [_splat(e), _splat(0)])
                w1 = plsc.load_gather(wv, [_splat(e), _splat(1)])
                for q in range(HID // L):
                    c0 = q * L
                    m = (w0 * (ub[b][e, pl.ds(c0, L)] + vb[b][e, pl.ds(c0, L)])
                         + w1 * (ub[b][e, pl.ds(HID + c0, L)] + vb[b][e, pl.ds(HID + c0, L)]))
                    msgv[e, pl.ds(c0, L)] = m
            pltpu.sync_copy(msgv, accsp.at[dstv], add=True)
        return 0

    lax.fori_loop(0, NCH // 2, pair, 0)
    plsc.subcore_barrier()
    rs = pl.ds(sidx * RPS, RPS)

    @pl.when(cidx == 0)
    def _():
        pltpu.sync_copy(accsp.at[rs], o0_h.at[rs])

    @pl.when(cidx == 1)
    def _():
        pltpu.sync_copy(accsp.at[rs], o1_h.at[rs])

  return _p2


def _make_p3(mesh):
  @functools.partial(
    pl.kernel,
    out_type=jax.ShapeDtypeStruct((EP, HID), f32),
    mesh=mesh,
    compiler_params=pltpu.CompilerParams(needs_layout_passes=False, use_tc_tiling_on_sc=False),
    scratch_types=[
        pltpu.VMEM((EPW,), i32),
        pltpu.VMEM((EPW,), i32),
        pltpu.VMEM((C, HID), f32),
        pltpu.VMEM((C, HID), f32),
        pltpu.VMEM((C, HID), f32),
        pltpu.VMEM((C, HID), f32),
        pltpu.VMEM((C, HID), f32),
        pltpu.VMEM((C, HID), f32),
        pltpu.VMEM((C, HID), f32),
        pltpu.VMEM((C, HID), f32),
        pltpu.SemaphoreType.DMA,
        pltpu.SemaphoreType.DMA,
        pltpu.SemaphoreType.DMA,
        pltpu.SemaphoreType.DMA,
    ],
  )
  def _p3(src_h, dst_h, p_h, q_h, r_h, eo_h, srca, dsta,
          pbuf0, pbuf1, qbuf0, qbuf1, rbuf0, rbuf1, ob0, ob1,
          semg0, semg1, semo0, semo1):
    cidx = lax.axis_index("c")
    sidx = lax.axis_index("s")
    wid = sidx * NC + cidx
    ebase = wid * EPW
    pltpu.sync_copy(src_h.at[pl.ds(ebase, EPW)], srca)
    pltpu.sync_copy(dst_h.at[pl.ds(ebase, EPW)], dsta)

    pbuf = (pbuf0, pbuf1)
    qbuf = (qbuf0, qbuf1)
    rbuf = (rbuf0, rbuf1)
    semg = (semg0, semg1)

    def issue(b, ci):
        pltpu.async_copy(p_h.at[srca.at[pl.ds(ci * C, C)]], pbuf[b], semg[b])
        pltpu.async_copy(q_h.at[dsta.at[pl.ds(ci * C, C)]], qbuf[b], semg[b])
        pltpu.async_copy(r_h.at[pl.ds(ebase + ci * C, C)], rbuf[b], semg[b])

    def wait(b, ci):
        pltpu.make_async_copy(p_h.at[srca.at[pl.ds(ci * C, C)]], pbuf[b], semg[b]).wait()
        pltpu.make_async_copy(q_h.at[dsta.at[pl.ds(ci * C, C)]], qbuf[b], semg[b]).wait()
        pltpu.make_async_copy(r_h.at[pl.ds(ebase + ci * C, C)], rbuf[b], semg[b]).wait()

    issue(0, 0)

    def pair(p, _):
        for b in range(2):
            ci = 2 * p + b

            @pl.when(ci + 1 < NCH)
            def _():
                issue(1 - b, ci + 1)

            wait(b, ci)
            for e in range(C):
                for q in range(HID // L):
                    sl = pl.ds(q * L, L)
                    v = pbuf[b][e, sl] + qbuf[b][e, sl] + rbuf[b][e, sl]
                    ob0[e, sl] = jnp.maximum(v, 0.0)
            pltpu.sync_copy(ob0, eo_h.at[pl.ds(ebase + ci * C, C)])
        return 0

    lax.fori_loop(0, NCH // 2, pair, 0)

  return _p3


@functools.lru_cache(maxsize=1)
def _sc_kernels():
    mesh = plsc.VectorSubcoreMesh(core_axis_name="c", subcore_axis_name="s",
                                  num_cores=NC, num_subcores=NS)
    return (_make_p1(mesh, 0), _make_p1(mesh, 2), _make_p2(mesh),
            _make_p3(mesh))


def _node_mat(Wn, Wen, Wa, din):
    wai, waj = Wa[0, :HID], Wa[0, HID:2 * HID]
    wen_a = Wen[:, :HID]
    cols = []
    for h in range(HEADS):
        wn_h = Wn[h * HID:(h + 1) * HID, :]
        cols.append(wn_h.T @ wen_a.T)
    sc = []
    for h in range(HEADS):
        wn_h = Wn[h * HID:(h + 1) * HID, :]
        sc.append((wn_h.T @ wai)[:, None])
    for h in range(HEADS):
        wn_h = Wn[h * HID:(h + 1) * HID, :]
        sc.append((wn_h.T @ waj)[:, None])
    pad = jnp.zeros((din, 256 - 2 * HID - 4), f32)
    return jnp.concatenate(cols + sc + [pad], axis=1)


def _edge_mat(We1, Wen1, Wa1, We2, Wen2, Wa2, Weu2):
    cols = []
    for We, Wen in ((We1, Wen1), (We2, Wen2)):
        wen_b = Wen[:, HID:]
        for h in range(HEADS):
            we_h = We[h * HID:(h + 1) * HID, :]
            cols.append(we_h.T @ wen_b.T)
    ae = []
    for We, Wa in ((We1, Wa1), (We2, Wa2)):
        wae = Wa[0, 2 * HID:]
        for h in range(HEADS):
            we_h = We[h * HID:(h + 1) * HID, :]
            ae.append((we_h.T @ wae)[:, None])
    ae.append(jnp.zeros((D_EDGE, 4), f32))
    we2_mean = 0.5 * (We2[:HID, :] + We2[HID:, :])
    rmat = we2_mean.T @ Weu2[:, 2 * HID:].T
    pad = jnp.zeros((D_EDGE, 384 - 256 - 8 - HID), f32)
    return jnp.concatenate(cols + ae + [rmat, pad], axis=1)


def kernel(x, edge_index, edge_attr, Wn1, We1, Wa1, Wen1, Weu1,
           Wn2, We2, Wa2, Wen2, Weu2):
    src = edge_index[0].astype(i32)
    dst = edge_index[1].astype(i32)
    loop = jnp.arange(N, dtype=i32)
    padi = jnp.full((EP - E - N,), N, i32)
    srcp = jnp.concatenate([src, loop, padi])
    dstp = jnp.concatenate([dst, loop, padi])
    x_pad = jnp.pad(x, ((0, NP - N), (0, 0)))
    ea_pad = jnp.pad(edge_attr, ((0, EP - E), (0, 0)))
    z2 = jnp.zeros((RPS, 2), f32)
    z64 = jnp.zeros((RPS, HID), f32)

    bn1 = _node_mat(Wn1, Wen1, Wa1, D_NODE)
    bn2 = _node_mat(Wn2, Wen2, Wa2, HID)
    be = _edge_mat(We1, Wen1, Wa1, We2, Wen2, Wa2, Weu2)
    wab = jnp.concatenate([Weu2[:, :HID].T, Weu2[:, HID:2 * HID].T], axis=1)

    p1l1, p1l2, p2, p3 = _sc_kernels()

    u1, sc1 = _tc_node(x_pad, bn1)
    v1, v2, ae8, rm = _tc_edge(ea_pad, be)

    ex1, d10, d11 = p1l1(srcp, dstp, ae8, sc1.reshape(-1), z2)
    ne10, ne11 = p2(srcp, dstp, ex1, d10.reshape(-1), d11.reshape(-1), u1, v1, z64)
    u2, sc2 = _tc_mid(ne10, ne11, bn2)
    ex2, d20, d21 = p1l2(srcp, dstp, ae8, sc2.reshape(-1), z2)
    ne20, ne21 = p2(srcp, dstp, ex2, d20.reshape(-1), d21.reshape(-1), u2, v2, z64)
    xo, pt, qt = _tc_fin(ne20, ne21, wab)
    eo = p3(srcp, dstp, pt, qt, rm)

    return xo[:N], eo[:E]
